# baseline (speedup 1.0000x reference)
import jax
import jax.numpy as jnp
from jax import lax
from jax.experimental import pallas as pl
from jax.experimental.pallas import tpu as pltpu
from jax.experimental.pallas import tpu_sc as plsc

_B = 16384
_F = 26
_E = 16
_N = _B * _F

_NC = 2
_NS = 16
_NW = _NC * _NS
_PER_W = _N // _NW
_CHUNK = 1024
_NCHUNK = _PER_W // _CHUNK


def _sc_body(ids_hbm, vals_hbm, w_hbm, out_hbm, idx_v, val_v, rows_v, sem):
    wid = lax.axis_index("s") * _NC + lax.axis_index("c")
    base = wid * _PER_W

    def chunk_body(c, carry):
        off = base + c * _CHUNK
        pltpu.sync_copy(ids_hbm.at[pl.ds(off, _CHUNK)], idx_v)
        pltpu.sync_copy(vals_hbm.at[pl.ds(off, _CHUNK)], val_v)
        pltpu.async_copy(w_hbm.at[idx_v], rows_v, sem).wait()

        def scale_body(g, carry2):
            r0 = g * 16
            v16 = val_v[pl.ds(r0, 16)]
            for j in range(16):
                rows_v[r0 + j, :] = rows_v[r0 + j, :] * v16[j]
            return carry2

        lax.fori_loop(0, _CHUNK // 16, scale_body, 0)
        pltpu.sync_copy(rows_v, out_hbm.at[pl.ds(off, _CHUNK)])
        return carry

    lax.fori_loop(0, _NCHUNK, chunk_body, 0)


def kernel(id, value, W):
    ids_flat = id.reshape(_N).astype(jnp.int32)
    vals_flat = value.reshape(_N)
    mesh = plsc.VectorSubcoreMesh(core_axis_name="c", subcore_axis_name="s")
    out = pl.kernel(
        _sc_body,
        mesh=mesh,
        compiler_params=pltpu.CompilerParams(use_tc_tiling_on_sc=False),
        out_type=jax.ShapeDtypeStruct((_N, _E), jnp.float32),
        scratch_types=[
            pltpu.VMEM((_CHUNK,), jnp.int32),
            pltpu.VMEM((_CHUNK,), jnp.float32),
            pltpu.VMEM((_CHUNK, _E), jnp.float32),
            pltpu.SemaphoreType.DMA,
        ],
    )(ids_flat, vals_flat, W)
    return out.reshape(_B, _F, _E)


# speedup vs baseline: 1.4698x; 1.4698x over previous
import jax
import jax.numpy as jnp
from jax import lax
from jax.experimental import pallas as pl
from jax.experimental.pallas import tpu as pltpu
from jax.experimental.pallas import tpu_sc as plsc

_B = 16384
_F = 26
_E = 16

_NC = 2
_NS = 16
_NW = _NC * _NS
_BW = _B // _NW


def _sc_body(idT_hbm, valT_hbm, w_hbm, out_hbm, idx_v, val_v, rows_v, obuf_v, sem):
    wid = lax.axis_index("s") * _NC + lax.axis_index("c")
    b0 = wid * _BW
    lanes = lax.iota(jnp.int32, 16)

    def f_body(f, carry):
        pltpu.sync_copy(idT_hbm.at[f, pl.ds(b0, _BW)], idx_v)
        pltpu.sync_copy(valT_hbm.at[f, pl.ds(b0, _BW)], val_v)
        pltpu.async_copy(w_hbm.at[idx_v], rows_v, sem).wait()

        def g_body(g, carry2):
            bidx = g * 16 + lanes
            v16 = val_v[pl.ds(g * 16, 16)]
            for e in range(_E):
                col = jnp.full((16,), e, jnp.int32)
                vals = plsc.load_gather(rows_v, [bidx, col])
                obuf_v[e, pl.ds(g * 16, 16)] = vals * v16
            return carry2

        lax.fori_loop(0, _BW // 16, g_body, 0)
        pltpu.sync_copy(obuf_v, out_hbm.at[f, :, pl.ds(b0, _BW)])
        return carry

    lax.fori_loop(0, _F, f_body, 0)


def kernel(id, value, W):
    idT = id.T
    valT = value.T
    mesh = plsc.VectorSubcoreMesh(core_axis_name="c", subcore_axis_name="s")
    outT = pl.kernel(
        _sc_body,
        mesh=mesh,
        compiler_params=pltpu.CompilerParams(use_tc_tiling_on_sc=False, needs_layout_passes=False),
        out_type=jax.ShapeDtypeStruct((_F, _E, _B), jnp.float32),
        scratch_types=[
            pltpu.VMEM((_BW,), jnp.int32),
            pltpu.VMEM((_BW,), jnp.float32),
            pltpu.VMEM((_BW, _E), jnp.float32),
            pltpu.VMEM((_E, _BW), jnp.float32),
            pltpu.SemaphoreType.DMA,
        ],
    )(idT, valT, W)
    return outT.transpose(2, 0, 1)
